# baseline (speedup 1.0000x reference)
import jax
import jax.numpy as jnp
from jax import lax
from jax.experimental import pallas as pl
from jax.experimental.pallas import tpu as pltpu
from jax.experimental.pallas import tpu_sc as plsc

NUM_EMBEDDINGS = 1024
EMBEDDING_DIM = 64
COMMITMENT_COST = 0.25

_T = 1024
_N_TOKENS = 32 * 1024
_GRID = _N_TOKENS // _T
_INV_N = 1.0 / (_N_TOKENS * EMBEDDING_DIM)

_NC = 2
_NS = 16
_NW = _NC * _NS
_BPW = _N_TOKENS // _NW
_CH = 128
_PIECES = _BPW // _CH
_DPAD = 128

_T2 = 4096


def _vq_tc_kernel(z_ref, e_ref, idx_ref, loss_ref, epad_ref):
    i = pl.program_id(0)
    z = z_ref[...]
    e = e_ref[...]
    zz = jnp.sum(z * z, axis=1, keepdims=True)
    e2 = jnp.sum(e * e, axis=1)
    mm = jax.lax.dot_general(
        z, e, dimension_numbers=(((1,), (1,)), ((), ())),
        preferred_element_type=jnp.float32)
    d = zz - 2.0 * mm + e2[None, :]
    dmin = jnp.min(d, axis=1, keepdims=True)
    iota = jax.lax.broadcasted_iota(jnp.int32, d.shape, 1)
    idx = jnp.min(jnp.where(d == dmin, iota, NUM_EMBEDDINGS), axis=1)
    idx_ref[...] = idx
    part = jnp.sum(dmin, keepdims=True)

    @pl.when(i == 0)
    def _():
        loss_ref[...] = part
        epad_ref[...] = jnp.concatenate(
            [e, jnp.zeros((NUM_EMBEDDINGS, _DPAD - EMBEDDING_DIM),
                          jnp.float32)], axis=1)

    @pl.when(i > 0)
    def _():
        loss_ref[...] += part

    @pl.when(i == _GRID - 1)
    def _():
        m = loss_ref[...] * _INV_N
        loss_ref[...] = m + COMMITMENT_COST * m


def _sc_gather(e_hbm, idx_hbm, qpad_hbm, idx_v, rows0, rows1, g0, g1, o0, o1):
    wid = lax.axis_index("s") * _NC + lax.axis_index("c")
    rows = (rows0, rows1)
    gsem = (g0, g1)
    osem = (o0, o1)
    pltpu.sync_copy(idx_hbm.at[pl.ds(wid * _BPW, _BPW)], idx_v)

    def start_gather(p):
        b = p % 2
        pltpu.async_copy(
            e_hbm.at[idx_v.at[pl.ds(p * _CH, _CH)]], rows[b], gsem[b])

    def start_out(p):
        b = p % 2
        pltpu.async_copy(
            rows[b], qpad_hbm.at[pl.ds(wid * _BPW + p * _CH, _CH)], osem[b])

    start_gather(0)
    for p in range(_PIECES):
        b = p % 2
        if p + 1 < _PIECES:
            if p + 1 >= 2:
                pltpu.make_async_copy(
                    rows[(p + 1) % 2],
                    qpad_hbm.at[pl.ds(wid * _BPW + (p - 1) * _CH, _CH)],
                    osem[(p + 1) % 2]).wait()
            start_gather(p + 1)
        pltpu.make_async_copy(
            e_hbm.at[idx_v.at[pl.ds(p * _CH, _CH)]], rows[b], gsem[b]).wait()
        start_out(p)
    for p in (_PIECES - 2, _PIECES - 1):
        b = p % 2
        pltpu.make_async_copy(
            rows[b], qpad_hbm.at[pl.ds(wid * _BPW + p * _CH, _CH)],
            osem[b]).wait()


_sc_call = pl.kernel(
    _sc_gather,
    out_type=jax.ShapeDtypeStruct((_N_TOKENS, _DPAD), jnp.float32),
    mesh=plsc.VectorSubcoreMesh(core_axis_name="c", subcore_axis_name="s"),
    scratch_types=[
        pltpu.VMEM((_BPW,), jnp.int32),
        pltpu.VMEM((_CH, _DPAD), jnp.float32),
        pltpu.VMEM((_CH, _DPAD), jnp.float32),
        pltpu.SemaphoreType.DMA,
        pltpu.SemaphoreType.DMA,
        pltpu.SemaphoreType.DMA,
        pltpu.SemaphoreType.DMA,
    ],
)


def _st_kernel(z_ref, qpad_ref, out_ref):
    z = z_ref[...]
    q = qpad_ref[:, :EMBEDDING_DIM]
    out_ref[...] = z + (q - z)


def kernel(inputs, embed_weight):
    flat = inputs.reshape(-1, EMBEDDING_DIM)
    idx, loss2, e_pad = pl.pallas_call(
        _vq_tc_kernel,
        grid=(_GRID,),
        in_specs=[
            pl.BlockSpec((_T, EMBEDDING_DIM), lambda i: (i, 0)),
            pl.BlockSpec((NUM_EMBEDDINGS, EMBEDDING_DIM), lambda i: (0, 0)),
        ],
        out_specs=[
            pl.BlockSpec((_T,), lambda i: (i,)),
            pl.BlockSpec((1, 1), lambda i: (0, 0)),
            pl.BlockSpec((NUM_EMBEDDINGS, _DPAD), lambda i: (0, 0)),
        ],
        out_shape=[
            jax.ShapeDtypeStruct((_N_TOKENS,), jnp.int32),
            jax.ShapeDtypeStruct((1, 1), jnp.float32),
            jax.ShapeDtypeStruct((NUM_EMBEDDINGS, _DPAD), jnp.float32),
        ],
    )(flat, embed_weight)
    q_pad = _sc_call(e_pad, idx)
    qst = pl.pallas_call(
        _st_kernel,
        grid=(_N_TOKENS // _T2,),
        in_specs=[
            pl.BlockSpec((_T2, EMBEDDING_DIM), lambda i: (i, 0)),
            pl.BlockSpec((_T2, _DPAD), lambda i: (i, 0)),
        ],
        out_specs=pl.BlockSpec((_T2, EMBEDDING_DIM), lambda i: (i, 0)),
        out_shape=jax.ShapeDtypeStruct((_N_TOKENS, EMBEDDING_DIM), jnp.float32),
    )(flat, q_pad)
    return (qst.reshape(inputs.shape), loss2[0, 0], idx)
